# Initial kernel scaffold; baseline (speedup 1.0000x reference)
import jax
import jax.numpy as jnp
from jax.experimental import pallas as pl


def kernel(feat, edge_index, e, snorm_n, snorm_e, W_emb, b_emb, Wp, bp, Wr, br):
    raise NotImplementedError("write your pallas kernel here")


# speedup vs baseline: 57.2360x; 57.2360x over previous
import functools

import jax
import jax.numpy as jnp
from jax import lax
from jax.experimental import pallas as pl
from jax.experimental.pallas import tpu as pltpu
from jax.experimental.pallas import tpu_sc as plsc

N = 10000
E = 320000
DIM = 128
L = 4
NC = 10

NTILES = 16
SLICE = 640
NPAD = NTILES * SLICE
CHUNK = 128
CPT = 160
EPT = CPT * CHUNK
EPAD = NTILES * EPT
NPADROWS = NPAD - N
GRP = 8


def _sc_propagate(dst_flat, dst_grp, src_grp):
    mesh = plsc.VectorSubcoreMesh(core_axis_name="c", subcore_axis_name="s")

    @functools.partial(
        pl.kernel,
        out_type=jax.ShapeDtypeStruct((L, NPAD), jnp.float32),
        mesh=mesh,
        compiler_params=pltpu.CompilerParams(needs_layout_passes=False),
        scratch_types=[
            pltpu.VMEM((EPT,), jnp.int32),
            pltpu.VMEM((CPT, CHUNK), jnp.int32),
            pltpu.VMEM((CPT, CHUNK), jnp.int32),
            pltpu.VMEM((EPT,), jnp.float32),
            pltpu.VMEM((NPAD,), jnp.float32),
            pltpu.VMEM((SLICE,), jnp.float32),
            pltpu.VMEM((SLICE,), jnp.float32),
            pltpu.VMEM((SLICE,), jnp.float32),
            pltpu.VMEM((SLICE,), jnp.float32),
            pltpu.VMEM_SHARED((NPAD,), jnp.float32),
            pltpu.VMEM_SHARED((NPAD,), jnp.float32),
            pltpu.SemaphoreType.DMA,
        ],
    )
    def k(dst_h, dstg_h, src_h, vout_h, dst_i, dst_r, src_i, upd, w_full,
          n_sl, v_sl, a_sl, z_sl, acc_sh, w_sh, sem):
        c = lax.axis_index("c")
        s = lax.axis_index("s")

        @pl.when(c == 0)
        def _body():
            base = s * SLICE
            ones16 = jnp.full((16,), 1.0, jnp.float32)

            pltpu.sync_copy(dst_h.at[s], dst_i)
            pltpu.sync_copy(dstg_h.at[s], dst_r)
            pltpu.sync_copy(src_h.at[s], src_i)

            @pl.loop(0, SLICE // 16)
            def _(i):
                z_sl[pl.ds(i * 16, 16)] = jnp.zeros((16,), jnp.float32)

            pltpu.sync_copy(z_sl, acc_sh.at[pl.ds(base, SLICE)])

            @pl.loop(0, EPT // 16)
            def _(i):
                upd[pl.ds(i * 16, 16)] = ones16

            plsc.subcore_barrier()

            @pl.loop(0, CPT // GRP)
            def _(g):
                descs = []
                for t in range(GRP):
                    j = g * GRP + t
                    descs.append(pltpu.async_copy(
                        upd.at[pl.ds(j * CHUNK, CHUNK)],
                        acc_sh.at[dst_r.at[j]], sem, add=True))
                for dsc in descs:
                    dsc.wait()

            plsc.subcore_barrier()

            pltpu.sync_copy(acc_sh.at[pl.ds(base, SLICE)], a_sl)

            @pl.loop(0, SLICE // 16)
            def _(i):
                d = jnp.maximum(a_sl[pl.ds(i * 16, 16)], 1.0)
                y = 1.0 / d
                hd = 0.5 * d
                for _it in range(22):
                    y = y * (1.5 - hd * y * y)
                n_sl[pl.ds(i * 16, 16)] = y
                v_sl[pl.ds(i * 16, 16)] = ones16

            pltpu.sync_copy(z_sl, acc_sh.at[pl.ds(base, SLICE)])
            pltpu.sync_copy(n_sl, w_sh.at[pl.ds(base, SLICE)])
            plsc.subcore_barrier()

            for m in range(L):
                pltpu.sync_copy(w_sh, w_full)

                @pl.loop(0, CPT // GRP)
                def _(g):
                    descs = []
                    for t in range(GRP):
                        j = g * GRP + t
                        for k8 in range(CHUNK // 16):
                            idxv = dst_i[pl.ds(j * CHUNK + k8 * 16, 16)]
                            vals = plsc.load_gather(w_full, [idxv])
                            upd[pl.ds(j * CHUNK + k8 * 16, 16)] = vals
                        descs.append(pltpu.async_copy(
                            upd.at[pl.ds(j * CHUNK, CHUNK)],
                            acc_sh.at[src_i.at[j]], sem, add=True))
                    for dsc in descs:
                        dsc.wait()

                plsc.subcore_barrier()

                pltpu.sync_copy(acc_sh.at[pl.ds(base, SLICE)], a_sl)
                pltpu.sync_copy(z_sl, acc_sh.at[pl.ds(base, SLICE)])

                @pl.loop(0, SLICE // 16)
                def _(i):
                    vv = v_sl[pl.ds(i * 16, 16)]
                    nn = n_sl[pl.ds(i * 16, 16)]
                    aw = a_sl[pl.ds(i * 16, 16)]
                    ww = w_full[pl.ds(base + i * 16, 16)]
                    vv = vv + nn * (ww + aw)
                    gidx = base + i * 16 + lax.iota(jnp.int32, 16)
                    vv = jnp.where(gidx < N, vv, 0.0)
                    v_sl[pl.ds(i * 16, 16)] = vv
                    a_sl[pl.ds(i * 16, 16)] = nn * vv

                pltpu.sync_copy(v_sl, vout_h.at[m, pl.ds(base, SLICE)])
                if m < L - 1:
                    pltpu.sync_copy(a_sl, w_sh.at[pl.ds(base, SLICE)])
                    plsc.subcore_barrier()

    return k(dst_flat, dst_grp, src_grp)


def _tc_vmatmul(vmat, feat_p):
    BN = 512
    grid = NPAD // BN

    def kern(v_ref, f_ref, m_ref):
        i = pl.program_id(0)

        @pl.when(i == 0)
        def _():
            m_ref[...] = jnp.zeros_like(m_ref)

        vb = v_ref[...]
        fb = f_ref[...]
        vh = vb.astype(jnp.bfloat16).astype(jnp.float32)
        vl = vb - vh
        part = (jnp.dot(vh, fb, preferred_element_type=jnp.float32)
                + jnp.dot(vl, fb, preferred_element_type=jnp.float32))
        colsum = jnp.sum(fb, axis=0, keepdims=True)
        m_ref[...] += jnp.concatenate(
            [colsum, part, jnp.zeros((3, DIM), jnp.float32)], axis=0)

    return pl.pallas_call(
        kern,
        grid=(grid,),
        in_specs=[
            pl.BlockSpec((L, BN), lambda i: (0, i)),
            pl.BlockSpec((BN, DIM), lambda i: (i, 0)),
        ],
        out_specs=pl.BlockSpec((8, DIM), lambda i: (0, 0)),
        out_shape=jax.ShapeDtypeStruct((8, DIM), jnp.float32),
    )(vmat, feat_p)


def _tc_readout(m8, vmat, W_emb, b_emb, Wp, bp, Wr, br):
    def kern(m_ref, v_ref, we_ref, be_ref, wp_ref, bp_ref, wr_ref, br_ref,
             out_ref):
        svec = jnp.sum(v_ref[...], axis=1)
        s = [jnp.float32(N)] + [svec[m] for m in range(L)]
        we = we_ref[...]
        be = be_ref[...]
        m8 = m_ref[...]
        rbk = lambda x: x.astype(jnp.bfloat16).astype(jnp.float32)

        def hidot(a, b):
            ah = rbk(a)
            return (jnp.dot(ah, b, preferred_element_type=jnp.float32)
                    + jnp.dot(a - ah, b, preferred_element_type=jnp.float32))

        Z = [hidot(m8[m:m + 1], we.T) + s[m] * be[None, :]
             for m in range(L + 1)]
        wr0 = wr_ref[0]
        score = jnp.dot(rbk(Z[0]), wr0.T, preferred_element_type=jnp.float32) \
            + br_ref[0][None, :]
        for i in range(1, L + 1):
            wp = wp_ref[i - 1]
            bpv = bp_ref[i - 1]
            Z = [hidot(Z[m + 1], wp.T) + s[m] * bpv[None, :]
                 for m in range(L + 1 - i)]
            score = score + jnp.dot(rbk(Z[0]), wr_ref[i].T,
                                    preferred_element_type=jnp.float32) \
                + br_ref[i][None, :]
        out_ref[...] = score

    return pl.pallas_call(
        kern,
        out_shape=jax.ShapeDtypeStruct((1, NC), jnp.float32),
    )(m8, vmat, W_emb, b_emb, Wp, bp, Wr, br)


def kernel(feat, edge_index, e, snorm_n, snorm_e, W_emb, b_emb, Wp, bp, Wr,
           br):
    src = edge_index[0]
    dst = edge_index[1]
    pad = (N + (jnp.arange(EPAD - E, dtype=jnp.int32) % NPADROWS))
    dst_p = jnp.concatenate([dst, pad])
    src_p = jnp.concatenate([src, pad])
    dst_flat = dst_p.reshape(NTILES, EPT)
    dst_grp = dst_p.reshape(NTILES, CPT, CHUNK)
    src_grp = src_p.reshape(NTILES, CPT, CHUNK)
    rb = lambda x: x.astype(jnp.bfloat16).astype(jnp.float32)
    feat_p = jnp.pad(rb(feat), ((0, NPAD - N), (0, 0)))
    W_emb, Wp, Wr = rb(W_emb), rb(Wp), rb(Wr)

    vmat = _sc_propagate(dst_flat, dst_grp, src_grp)
    m8 = _tc_vmatmul(vmat, feat_p)
    return _tc_readout(m8, vmat, W_emb, b_emb, Wp, bp, Wr, br)
